# baseline (speedup 1.0000x reference)
import functools

import jax
import jax.numpy as jnp
from jax.experimental import pallas as pl

B, S, D, E, K = 4, 8192, 768, 8, 2
T = 2048


def _router_block(x_ref, w_ref, weights_ref, topw_ref, tope_ref):
    x = x_ref[...]
    w = w_ref[...]
    logits = jax.lax.dot_general(
        x, w, (((1,), (0,)), ((), ())), preferred_element_type=jnp.float32
    )
    m = jnp.max(logits, axis=-1, keepdims=True)
    ex = jnp.exp(logits - m)
    denom = jnp.sum(ex, axis=-1, keepdims=True)
    weights_ref[...] = ex / denom

    ids = jax.lax.broadcasted_iota(jnp.int32, logits.shape, 1)
    big = jnp.int32(E)
    m1 = jnp.max(logits, axis=-1, keepdims=True)
    id1 = jnp.min(jnp.where(logits == m1, ids, big), axis=-1, keepdims=True)
    neg = jnp.float32(-jnp.inf)
    l2 = jnp.where(ids == id1, neg, logits)
    m2 = jnp.max(l2, axis=-1, keepdims=True)
    id2 = jnp.min(jnp.where(l2 == m2, ids, big), axis=-1, keepdims=True)

    e2 = jnp.exp(m2 - m1)
    w1 = 1.0 / (1.0 + e2)
    w2 = 1.0 - w1
    topw_ref[...] = jnp.concatenate([w1, w2], axis=-1)
    tope_ref[...] = jnp.concatenate([id1, id2], axis=-1)


@jax.jit
def kernel(x, W):
    N = B * S
    xf = x.reshape(N, D)
    grid = (N // T,)
    weights, topw, tope = pl.pallas_call(
        _router_block,
        grid=grid,
        in_specs=[
            pl.BlockSpec((T, D), lambda i: (i, 0)),
            pl.BlockSpec((D, E), lambda i: (0, 0)),
        ],
        out_specs=[
            pl.BlockSpec((T, E), lambda i: (i, 0)),
            pl.BlockSpec((T, K), lambda i: (i, 0)),
            pl.BlockSpec((T, K), lambda i: (i, 0)),
        ],
        out_shape=[
            jax.ShapeDtypeStruct((N, E), jnp.float32),
            jax.ShapeDtypeStruct((N, K), jnp.float32),
            jax.ShapeDtypeStruct((N, K), jnp.int32),
        ],
    )(xf, W)
    return (
        weights.reshape(B, S, E),
        topw.reshape(B, S, K),
        tope.reshape(B, S, K),
    )


# speedup vs baseline: 2.4764x; 2.4764x over previous
import jax
import jax.numpy as jnp
from jax.experimental import pallas as pl

B, S, D, E, K = 4, 8192, 768, 8, 2
T = 2048


def _router_block(x_ref, wt_ref, weights_t_ref, topw_t_ref, tope_t_ref):
    x = x_ref[...]
    wt = wt_ref[...]
    logits_t = jax.lax.dot_general(
        wt, x, (((1,), (1,)), ((), ())), preferred_element_type=jnp.float32
    )

    ids = jax.lax.broadcasted_iota(jnp.int32, logits_t.shape, 0)
    big = jnp.int32(E)
    neg = jnp.float32(-jnp.inf)

    m1 = jnp.max(logits_t, axis=0, keepdims=True)
    id1 = jnp.min(jnp.where(logits_t == m1, ids, big), axis=0, keepdims=True)
    l2 = jnp.where(ids == id1, neg, logits_t)
    m2 = jnp.max(l2, axis=0, keepdims=True)
    id2 = jnp.min(jnp.where(l2 == m2, ids, big), axis=0, keepdims=True)

    ex = jnp.exp(logits_t - m1)
    denom = jnp.sum(ex, axis=0, keepdims=True)
    weights_t_ref[...] = ex / denom

    e2 = jnp.exp(m2 - m1)
    w1 = 1.0 / (1.0 + e2)
    topw_t_ref[...] = jnp.concatenate([w1, 1.0 - w1], axis=0)
    tope_t_ref[...] = jnp.concatenate([id1, id2], axis=0)


@jax.jit
def kernel(x, W):
    N = B * S
    xf = x.reshape(N, D)
    wt = W.T
    grid = (N // T,)
    weights_t, topw_t, tope_t = pl.pallas_call(
        _router_block,
        grid=grid,
        in_specs=[
            pl.BlockSpec((T, D), lambda i: (i, 0)),
            pl.BlockSpec((E, D), lambda i: (0, 0)),
        ],
        out_specs=[
            pl.BlockSpec((E, T), lambda i: (0, i)),
            pl.BlockSpec((K, T), lambda i: (0, i)),
            pl.BlockSpec((K, T), lambda i: (0, i)),
        ],
        out_shape=[
            jax.ShapeDtypeStruct((E, N), jnp.float32),
            jax.ShapeDtypeStruct((K, N), jnp.float32),
            jax.ShapeDtypeStruct((K, N), jnp.int32),
        ],
    )(xf, wt)
    return (
        weights_t.T.reshape(B, S, E),
        topw_t.T.reshape(B, S, K),
        tope_t.T.reshape(B, S, K),
    )


# speedup vs baseline: 2.5243x; 1.0194x over previous
import jax
import jax.numpy as jnp
from jax.experimental import pallas as pl

B, S, D, E, K = 4, 8192, 768, 8, 2
T = 4096


def _router_block(x_ref, wt_ref, weights_t_ref, topw_t_ref, tope_t_ref):
    x = x_ref[...]
    wt = wt_ref[...]
    logits_t = jax.lax.dot_general(
        wt, x, (((1,), (1,)), ((), ())), preferred_element_type=jnp.float32
    )

    ids = jax.lax.broadcasted_iota(jnp.int32, logits_t.shape, 0)
    big = jnp.int32(E)
    neg = jnp.float32(-jnp.inf)

    m1 = jnp.max(logits_t, axis=0, keepdims=True)
    id1 = jnp.min(jnp.where(logits_t == m1, ids, big), axis=0, keepdims=True)
    l2 = jnp.where(ids == id1, neg, logits_t)
    m2 = jnp.max(l2, axis=0, keepdims=True)
    id2 = jnp.min(jnp.where(l2 == m2, ids, big), axis=0, keepdims=True)

    ex = jnp.exp(logits_t - m1)
    denom = jnp.sum(ex, axis=0, keepdims=True)
    weights_t_ref[...] = ex / denom

    e2 = jnp.exp(m2 - m1)
    w1 = 1.0 / (1.0 + e2)
    topw_t_ref[...] = jnp.concatenate([w1, 1.0 - w1], axis=0)
    tope_t_ref[...] = jnp.concatenate([id1, id2], axis=0)


@jax.jit
def kernel(x, W):
    N = B * S
    xf = x.reshape(N, D)
    wt = W.T
    grid = (N // T,)
    weights_t, topw_t, tope_t = pl.pallas_call(
        _router_block,
        grid=grid,
        in_specs=[
            pl.BlockSpec((T, D), lambda i: (i, 0)),
            pl.BlockSpec((E, D), lambda i: (0, 0)),
        ],
        out_specs=[
            pl.BlockSpec((E, T), lambda i: (0, i)),
            pl.BlockSpec((K, T), lambda i: (0, i)),
            pl.BlockSpec((K, T), lambda i: (0, i)),
        ],
        out_shape=[
            jax.ShapeDtypeStruct((E, N), jnp.float32),
            jax.ShapeDtypeStruct((K, N), jnp.float32),
            jax.ShapeDtypeStruct((K, N), jnp.int32),
        ],
    )(xf, wt)
    return (
        weights_t.T.reshape(B, S, E),
        topw_t.T.reshape(B, S, K),
        tope_t.T.reshape(B, S, K),
    )
